# baseline (speedup 1.0000x reference)
import functools

import jax
import jax.numpy as jnp
from jax import lax
from jax.experimental import pallas as pl
from jax.experimental.pallas import tpu as pltpu
from jax.experimental.pallas import tpu_sc as plsc

_GROUPS = 2
_NUM_VARS = 512
_VAR_DIM = 64
_OUT_DIM = _GROUPS * _VAR_DIM
_N_TILE = 256
_LANES = 16


def _logits_kernel(x_ref, w_ref, b_ref, idx_ref, probs_ref):
    i = pl.program_id(0)
    logits = jax.lax.dot_general(
        x_ref[:], w_ref[:],
        dimension_numbers=(((1,), (1,)), ((), ())),
        preferred_element_type=jnp.float32,
    ) + b_ref[:]
    psums = []
    ks = []
    for g in range(_GROUPS):
        lg = logits[:, g * _NUM_VARS:(g + 1) * _NUM_VARS]
        m = jnp.max(lg, axis=-1, keepdims=True)
        e = jnp.exp(lg - m)
        s = jnp.sum(e, axis=-1, keepdims=True)
        psums.append(jnp.sum(e / s, axis=0))
        idx = jax.lax.broadcasted_iota(jnp.int32, lg.shape, 1)
        k = jnp.min(jnp.where(lg == m, idx, _NUM_VARS), axis=-1, keepdims=True)
        ks.append(k + g * _NUM_VARS)
    idx_ref[:] = jnp.concatenate(ks, axis=1)
    psum = jnp.concatenate(psums).reshape(1, _GROUPS * _NUM_VARS)

    @pl.when(i == 0)
    def _():
        probs_ref[:] = psum

    @pl.when(i != 0)
    def _():
        probs_ref[:] = probs_ref[:] + psum


def _make_gather(n_tokens):
    info = plsc.get_sparse_core_info()
    nc, ns = info.num_cores, info.num_subcores
    nw = nc * ns
    tpw = n_tokens // nw
    mesh = plsc.VectorSubcoreMesh(core_axis_name="c", subcore_axis_name="s")

    @functools.partial(
        pl.kernel, mesh=mesh,
        out_type=jax.ShapeDtypeStruct((n_tokens, _OUT_DIM), jnp.float32),
        scratch_types=[
            pltpu.VMEM((tpw,), jnp.int32),
            pltpu.VMEM((tpw,), jnp.int32),
            pltpu.VMEM((tpw, _OUT_DIM), jnp.float32),
            pltpu.VMEM((tpw, _OUT_DIM), jnp.float32),
            pltpu.SemaphoreType.DMA,
        ],
    )
    def gather(cb_hbm, idx0_hbm, idx1_hbm, out_hbm,
               idx0_v, idx1_v, rows0_v, rows1_v, sem):
        wid = lax.axis_index("s") * nc + lax.axis_index("c")
        base = wid * tpw
        pltpu.sync_copy(idx0_hbm.at[pl.ds(base, tpw)], idx0_v)
        pltpu.sync_copy(idx1_hbm.at[pl.ds(base, tpw)], idx1_v)
        c0 = pltpu.async_copy(cb_hbm.at[idx0_v], rows0_v, sem)
        c1 = pltpu.async_copy(cb_hbm.at[idx1_v], rows1_v, sem)
        c0.wait()
        c1.wait()

        def row(i, _):
            def chunk(j, _):
                s = pl.ds(j * _LANES, _LANES)
                rows0_v[i, s] = rows0_v[i, s] + rows1_v[i, s]
                return 0
            return lax.fori_loop(0, _OUT_DIM // _LANES, chunk, 0)

        lax.fori_loop(0, tpw, row, 0)
        pltpu.sync_copy(rows0_v, out_hbm.at[pl.ds(base, tpw)])

    return gather


def kernel(x, W, b, codebook):
    bsz, t, d = x.shape
    n = bsz * t
    flat = x.reshape(n, d)
    cb = codebook.reshape(_GROUPS * _NUM_VARS, _VAR_DIM)
    cb_pad = jnp.concatenate(
        [jnp.pad(cb[:_NUM_VARS], ((0, 0), (0, _VAR_DIM))),
         jnp.pad(cb[_NUM_VARS:], ((0, 0), (_VAR_DIM, 0)))], axis=0)
    grid = n // _N_TILE
    idx, probs = pl.pallas_call(
        _logits_kernel,
        grid=(grid,),
        in_specs=[
            pl.BlockSpec((_N_TILE, d), lambda i: (i, 0)),
            pl.BlockSpec((_GROUPS * _NUM_VARS, d), lambda i: (0, 0)),
            pl.BlockSpec((1, _GROUPS * _NUM_VARS), lambda i: (0, 0)),
        ],
        out_specs=[
            pl.BlockSpec((_N_TILE, _GROUPS), lambda i: (i, 0)),
            pl.BlockSpec((1, _GROUPS * _NUM_VARS), lambda i: (0, 0)),
        ],
        out_shape=[
            jax.ShapeDtypeStruct((n, _GROUPS), jnp.int32),
            jax.ShapeDtypeStruct((1, _GROUPS * _NUM_VARS), jnp.float32),
        ],
    )(flat, W, b.reshape(1, -1))
    out = _make_gather(n)(cb_pad, idx[:, 0], idx[:, 1])
    avg_probs = (probs / n).reshape(_GROUPS, _NUM_VARS)
    return out.reshape(bsz, t, _OUT_DIM), avg_probs
